# baseline (speedup 1.0000x reference)
import functools

import jax
import jax.numpy as jnp
from jax import lax
from jax.experimental import pallas as pl
from jax.experimental.pallas import tpu as pltpu
from jax.experimental.pallas import tpu_sc as plsc

LEN = 200
BATCH = 1024
DIM = 64
OUT_DIM = 64
B = LEN * BATCH

NC = 2
NS = 16
NW = NC * NS
B_PER_W = B // NW
GCHUNK = 128
N_GCHUNKS = B_PER_W // GCHUNK

@functools.cache
def _make_sc_gather():
    mesh = plsc.VectorSubcoreMesh(core_axis_name="c", subcore_axis_name="s")

    @functools.partial(
        pl.kernel,
        mesh=mesh,
        compiler_params=pltpu.CompilerParams(use_tc_tiling_on_sc=False),
        out_type=jax.ShapeDtypeStruct((B, DIM), jnp.float32),
        scratch_types=[
            pltpu.VMEM((N_GCHUNKS, GCHUNK), jnp.int32),
            pltpu.VMEM((GCHUNK, DIM), jnp.float32),
            pltpu.SemaphoreType.DMA,
        ],
    )
    def _sc_gather(idx_hbm, table_hbm, out_hbm, idx_v, rows_v, sem):
        wid = lax.axis_index("s") * NC + lax.axis_index("c")
        base = wid * B_PER_W
        pltpu.sync_copy(idx_hbm.at[wid], idx_v)

        def body(j, carry):
            pltpu.async_copy(table_hbm.at[idx_v.at[j]], rows_v, sem).wait()
            pltpu.sync_copy(rows_v, out_hbm.at[pl.ds(base + j * GCHUNK, GCHUNK)])
            return carry

        lax.fori_loop(0, N_GCHUNKS, body, 0)

    return _sc_gather


ROWS_BLK = 2048


def _proj_body(x_ref, w_ref, b_ref, o_ref):
    acc = jnp.dot(x_ref[...], w_ref[...], preferred_element_type=jnp.float32)
    o_ref[...] = jnp.maximum(acc + b_ref[...], 0.0)


def _project(g, W_proj, b_proj):
    return pl.pallas_call(
        _proj_body,
        grid=(B // ROWS_BLK,),
        in_specs=[
            pl.BlockSpec((ROWS_BLK, DIM), lambda i: (i, 0)),
            pl.BlockSpec((DIM, OUT_DIM), lambda i: (0, 0)),
            pl.BlockSpec((1, OUT_DIM), lambda i: (0, 0)),
        ],
        out_specs=pl.BlockSpec((ROWS_BLK, OUT_DIM), lambda i: (i, 0)),
        out_shape=jax.ShapeDtypeStruct((B, OUT_DIM), jnp.float32),
    )(g, W_proj, b_proj.reshape(1, OUT_DIM))


def kernel(input, W_emb, W_proj, b_proj):
    idx = input.reshape(NW, N_GCHUNKS, GCHUNK).astype(jnp.int32)
    g = _make_sc_gather()(idx, W_emb)
    out = _project(g, W_proj, b_proj)
    return out.reshape(LEN, BATCH, OUT_DIM)


# speedup vs baseline: 1.0348x; 1.0348x over previous
import functools

import jax
import jax.numpy as jnp
from jax import lax
from jax.experimental import pallas as pl
from jax.experimental.pallas import tpu as pltpu
from jax.experimental.pallas import tpu_sc as plsc

LEN = 200
BATCH = 1024
DIM = 64
OUT_DIM = 64
B = LEN * BATCH

NC = 2
NS = 16
NW = NC * NS
B_PER_W = B // NW
GCHUNK = 128
N_GCHUNKS = B_PER_W // GCHUNK

NBUF = 5
NGROUP = N_GCHUNKS // NBUF


@functools.cache
def _make_sc_gather():
    mesh = plsc.VectorSubcoreMesh(core_axis_name="c", subcore_axis_name="s")

    @functools.partial(
        pl.kernel,
        mesh=mesh,
        compiler_params=pltpu.CompilerParams(use_tc_tiling_on_sc=False),
        out_type=jax.ShapeDtypeStruct((B, DIM), jnp.float32),
        scratch_types=[
            pltpu.VMEM((N_GCHUNKS, GCHUNK), jnp.int32),
            pltpu.VMEM((2, NBUF, GCHUNK, DIM), jnp.float32),
            pltpu.SemaphoreType.DMA,
            pltpu.SemaphoreType.DMA,
        ],
    )
    def _sc_gather(idx_hbm, table_hbm, out_hbm, idx_v, rows_v, gsem, wsem):
        wid = lax.axis_index("s") * NC + lax.axis_index("c")
        base = wid * B_PER_W
        pltpu.sync_copy(idx_hbm.at[wid], idx_v)

        def group(g, carry):
            p = lax.rem(g, 2)

            @pl.when(g >= 2)
            def _():
                for b in range(NBUF):
                    pltpu.make_async_copy(
                        rows_v.at[p, b], out_hbm.at[pl.ds(base, GCHUNK)], wsem
                    ).wait()

            descs = []
            for b in range(NBUF):
                j = g * NBUF + b
                descs.append(
                    pltpu.async_copy(table_hbm.at[idx_v.at[j]], rows_v.at[p, b], gsem)
                )
            for d in descs:
                d.wait()
            for b in range(NBUF):
                j = g * NBUF + b
                pltpu.async_copy(
                    rows_v.at[p, b], out_hbm.at[pl.ds(base + j * GCHUNK, GCHUNK)], wsem
                )
            return carry

        lax.fori_loop(0, NGROUP, group, 0)
        for p in range(2):
            for b in range(NBUF):
                pltpu.make_async_copy(
                    rows_v.at[p, b], out_hbm.at[pl.ds(base, GCHUNK)], wsem
                ).wait()

    return _sc_gather


ROWS_BLK = 2048


def _proj_body(x_ref, w_ref, b_ref, o_ref):
    acc = jnp.dot(x_ref[...], w_ref[...], preferred_element_type=jnp.float32)
    o_ref[...] = jnp.maximum(acc + b_ref[...], 0.0)


def _project(g, W_proj, b_proj):
    return pl.pallas_call(
        _proj_body,
        grid=(B // ROWS_BLK,),
        in_specs=[
            pl.BlockSpec((ROWS_BLK, DIM), lambda i: (i, 0)),
            pl.BlockSpec((DIM, OUT_DIM), lambda i: (0, 0)),
            pl.BlockSpec((1, OUT_DIM), lambda i: (0, 0)),
        ],
        out_specs=pl.BlockSpec((ROWS_BLK, OUT_DIM), lambda i: (i, 0)),
        out_shape=jax.ShapeDtypeStruct((B, OUT_DIM), jnp.float32),
    )(g, W_proj, b_proj.reshape(1, OUT_DIM))


def kernel(input, W_emb, W_proj, b_proj):
    idx = input.reshape(NW, N_GCHUNKS, GCHUNK).astype(jnp.int32)
    g = _make_sc_gather()(idx, W_emb)
    out = _project(g, W_proj, b_proj)
    return out.reshape(LEN, BATCH, OUT_DIM)
